# baseline (speedup 1.0000x reference)
import jax
import jax.numpy as jnp
from jax.experimental import pallas as pl
from jax.experimental.pallas import tpu as pltpu

_SCORE_THRESH = 0.05
_NMS_THRESH = 0.5
_DET = 300
_DETP = 304
_K = 1000
_KP = 1024
_N = 20000
_NP = 20480
_NB = 160
_B = 4
_L = 512
_TPAD = _KP + 128
_ONE_BITS = 0x3F800000
_HI = jax.lax.Precision.HIGHEST


def _gauss_params(cx, cy, w, h, ang):
    c, s = jnp.cos(ang), jnp.sin(ang)
    w2, h2 = (w * w) / 12.0, (h * h) / 12.0
    a = w2 * c * c + h2 * s * s
    b = w2 * s * s + h2 * c * c
    cc = (w2 - h2) * c * s
    return cx, cy, a, b, cc


def _body(feat_ref, scores_ref,
          ob_ref, os_ref, ol_ref,
          s4_ref, p4_ref, tab_ref, sel_ref, offs_ref, valid_ref, keep_ref):
    f32 = jnp.float32
    bf16 = jnp.bfloat16
    i32 = jnp.int32
    sc4 = scores_ref[...]
    key = jnp.where(sc4 > _SCORE_THRESH,
                    jax.lax.bitcast_convert_type(sc4, i32), i32(-1))
    gg = ((jax.lax.broadcasted_iota(i32, (_L, _L), 0) // 128)
          == (jax.lax.broadcasted_iota(i32, (_L, _L), 1) // 128)).astype(f32)
    dn_mv = (((1,), (0,)), ((), ()))

    def _gcount(mask):
        col = jnp.sum(mask.astype(f32), axis=0, keepdims=True)
        return jax.lax.dot_general(col, gg, dn_mv,
                                   preferred_element_type=f32)

    def _val_step(_, lohi):
        lo, hi = lohi
        d = jnp.maximum((hi - lo) // 4, 1)
        m1 = jnp.minimum(lo + d, hi)
        m2 = jnp.minimum(m1 + d, hi)
        m3 = jnp.minimum(m2 + d, hi)
        b1 = _gcount(key >= m1) >= _K
        b2 = _gcount(key >= m2) >= _K
        b3 = _gcount(key >= m3) >= _K
        lo = jnp.where(b3, m3, jnp.where(b2, m2, jnp.where(b1, m1, lo)))
        hi = jnp.where(b3, hi, jnp.where(b2, m3, jnp.where(b1, m2, m1)))
        return lo, hi

    tau, _ = jax.lax.fori_loop(
        0, 17, _val_step, (jnp.full((1, _L), -1, i32),
                           jnp.full((1, _L), _ONE_BITS, i32)))
    eq = key == tau
    m_need = f32(_K) - _gcount(key > tau)

    lstrict = (jax.lax.broadcasted_iota(i32, (_NB, _NB), 1)
               < jax.lax.broadcasted_iota(i32, (_NB, _NB), 0)).astype(f32)
    tri_s = ((jax.lax.broadcasted_iota(i32, (_L, _L), 0)
              < jax.lax.broadcasted_iota(i32, (_L, _L), 1))
             & ((jax.lax.broadcasted_iota(i32, (_L, _L), 0) // 128)
                == (jax.lax.broadcasted_iota(i32, (_L, _L), 1) // 128))
             ).astype(f32)
    eqf = eq.astype(f32)
    rowcnt = jax.lax.dot_general(eqf, gg, dn_mv,
                                 preferred_element_type=f32)
    rank2d = (jax.lax.dot_general(lstrict, rowcnt, dn_mv,
                                  preferred_element_type=f32)
              + jax.lax.dot_general(eqf, tri_s, dn_mv,
                                    preferred_element_type=f32))
    sel = ((key > tau) | (eq & (rank2d < m_need))).astype(f32)
    sel_ref[...] = sel

    cnts = jax.lax.dot_general(sel, gg, dn_mv,
                               preferred_element_type=f32)
    offs512 = jnp.minimum(jax.lax.dot_general(
        lstrict, cnts, dn_mv, preferred_element_type=f32), f32(_K))
    for i in range(_B):
        offs_ref[i] = offs512[:, 128 * i:128 * i + 1]
    tri = ((jax.lax.broadcasted_iota(i32, (_L, _L), 0)
            <= jax.lax.broadcasted_iota(i32, (_L, _L), 1))
           & ((jax.lax.broadcasted_iota(i32, (_L, _L), 0) // 128)
              == (jax.lax.broadcasted_iota(i32, (_L, _L), 1) // 128))
           ).astype(f32)
    riota = jax.lax.broadcasted_iota(i32, (128, _L), 0).astype(f32)
    dn = (((1,), (1,)), ((), ()))

    def _gather_step(t, carry):
        m = sel_ref[pl.ds(t, 1), :]
        csum = jax.lax.dot_general(m, tri, dn_mv,
                                   preferred_element_type=f32)
        g = jnp.where(m > 0.5, csum - 1.0, -1.0)
        ct = (riota == g).astype(f32)
        dt = feat_ref[:, pl.ds(t, 1), :].reshape(9, _L)
        for i in range(_B):
            compact = jax.lax.dot_general(
                ct[:, 128 * i:128 * (i + 1)], dt[:, 128 * i:128 * (i + 1)],
                dn, preferred_element_type=f32, precision=_HI)
            off = offs_ref[i, t, 0].astype(i32)
            tab_ref[i, pl.ds(off, 128), :] = compact
        return carry

    jax.lax.fori_loop(0, _NB, _gather_step, 0, unroll=16)

    def _build_step(img, carry):
        tab_ref[img, _K:, :] = jnp.zeros((_TPAD - _K, 9), f32)
        tab = tab_ref[img, 0:_KP, :]
        tabt = jnp.transpose(tab)
        cxc, cyc, wc, hc, ac = (tab[:, i:i + 1] for i in range(5))
        scc, labc, idxc = tab[:, 5:6], tab[:, 6:7], tab[:, 7:8]
        cxr, cyr, wr, hr, ar = (tabt[i:i + 1, :] for i in range(5))
        scr, labr, idxr = tabt[5:6, :], tabt[6:7, :], tabt[7:8, :]

        x1, y1, a1, b1, c1 = _gauss_params(cxc, cyc, wc, hc, ac)
        x2, y2, a2, b2, c2 = _gauss_params(cxr, cyr, wr, hr, ar)
        eps = 1e-7
        dx = x1 - x2
        dy = y1 - y2
        ab = (a1 + a2) * (b1 + b2) - (c1 + c2) ** 2
        t1 = 0.25 * ((a1 + a2) * dy * dy + (b1 + b2) * dx * dx) / (ab + eps)
        t2 = 0.5 * ((c1 + c2) * (-dx) * dy) / (ab + eps)
        d1 = jnp.clip(a1 * b1 - c1 * c1, eps, None)
        d2 = jnp.clip(a2 * b2 - c2 * c2, eps, None)
        t3 = 0.5 * jnp.log(ab / (4.0 * jnp.sqrt(d1 * d2) + eps) + eps)
        bd = jnp.clip(t1 + t2 + t3, eps, 100.0)
        hd = jnp.sqrt(1.0 - jnp.exp(-bd) + eps)
        iou = 1.0 - hd

        prio = (scc > scr) | ((scc == scr) & (idxc < idxr))
        p4_ref[img] = prio.astype(bf16)
        s4_ref[img] = ((iou > _NMS_THRESH) & (labc == labr)
                       & prio).astype(bf16)
        valid_ref[img] = (scr > _SCORE_THRESH).astype(bf16)
        return carry

    jax.lax.fori_loop(0, _B, _build_step, 0, unroll=2)

    valid4 = valid_ref[...]

    def _one_step(old):
        hits = [jax.lax.dot_general(old[i], s4_ref[i], dn_mv,
                                    preferred_element_type=f32
                                    ).reshape(1, 1, _KP)
                for i in range(_B)]
        hit4 = jnp.concatenate(hits, axis=0)
        return jnp.where(hit4 < 0.5, valid4, bf16(0))

    def _nms_cond(changed):
        return changed

    def _nms_step(_):
        old = keep_ref[...]
        new = _one_step(old)
        keep_ref[...] = new
        return jnp.sum(jnp.abs((new - old).astype(f32))) > 0.0

    keep_ref[...] = _one_step(_one_step(valid4))
    jax.lax.while_loop(_nms_cond, _nms_step, jnp.bool_(True))

    piota = jax.lax.broadcasted_iota(i32, (_DETP, _KP), 0).astype(f32)

    def _out_step(img, carry):
        kept = keep_ref[img]
        rank = jax.lax.dot_general(kept, p4_ref[img], dn_mv,
                                   preferred_element_type=f32)
        oh = ((piota == rank) & (kept > bf16(0.5))).astype(f32)
        tab = tab_ref[img, 0:_KP, :]
        out = jax.lax.dot_general(oh, tab, dn_mv,
                                  preferred_element_type=f32,
                                  precision=_HI)
        outt = jax.lax.dot_general(jnp.transpose(tab), oh, dn,
                                   preferred_element_type=f32,
                                   precision=_HI)
        ob_ref[pl.ds(img, 1), :, :] = out[:_DET, 0:5].reshape(1, _DET, 5)
        os_ref[pl.ds(img, 1), :] = outt[5:6, :_DET]
        ind = outt[8:9, :_DET]
        ol_ref[pl.ds(img, 1), :] = jnp.where(
            ind > 0.5, outt[6:7, :_DET], -1.0).astype(jnp.int32)
        return carry

    jax.lax.fori_loop(0, _B, _out_step, 0)


@jax.jit
def kernel(boxes, scores, labels):
    b = boxes.shape[0]
    f32 = jnp.float32
    pad = _NP - _N
    sc = jnp.pad(scores, ((0, 0), (0, pad)),
                 constant_values=-1.0).reshape(b, _NB, 128)
    sc = jnp.transpose(sc, (1, 0, 2)).reshape(_NB, b * 128)
    bx = jnp.pad(boxes, ((0, 0), (0, pad), (0, 0)))
    bx = jnp.transpose(bx, (0, 2, 1))
    scp = jnp.pad(scores, ((0, 0), (0, pad)))[:, None, :]
    lbp = jnp.pad(labels, ((0, 0), (0, pad))).astype(f32)[:, None, :]
    idx = jnp.broadcast_to(jnp.arange(_NP, dtype=f32), (b, 1, _NP))
    one = jnp.ones((b, 1, _NP), f32)
    feat = jnp.concatenate([bx, scp, lbp, idx, one],
                           axis=1).reshape(b, 9, _NB, 128)
    feat = jnp.transpose(feat, (1, 2, 0, 3)).reshape(9, _NB, b * 128)

    out = pl.pallas_call(
        _body,
        out_shape=[
            jax.ShapeDtypeStruct((b, _DET, 5), f32),
            jax.ShapeDtypeStruct((b, _DET), f32),
            jax.ShapeDtypeStruct((b, _DET), jnp.int32),
        ],
        scratch_shapes=[
            pltpu.VMEM((_B, _KP, _KP), jnp.bfloat16),
            pltpu.VMEM((_B, _KP, _KP), jnp.bfloat16),
            pltpu.VMEM((_B, _TPAD, 9), f32),
            pltpu.VMEM((_NB, _L), f32),
            pltpu.VMEM((_B, _NB, 1), f32),
            pltpu.VMEM((_B, 1, _KP), jnp.bfloat16),
            pltpu.VMEM((_B, 1, _KP), jnp.bfloat16),
        ],
    )(feat, sc)
    return tuple(out)


# speedup vs baseline: 1.0489x; 1.0489x over previous
import jax
import jax.numpy as jnp
from jax.experimental import pallas as pl
from jax.experimental.pallas import tpu as pltpu

_SCORE_THRESH = 0.05
_NMS_THRESH = 0.5
_DET = 300
_DETP = 304
_K = 1000
_KP = 1024
_N = 20000
_NP = 20480
_NB = 160
_B = 4
_L = 512
_TPAD = _KP + 128
_ONE_BITS = 0x3F800000
_HI = jax.lax.Precision.HIGHEST


def _gauss_params(cx, cy, w, h, ang):
    c, s = jnp.cos(ang), jnp.sin(ang)
    w2, h2 = (w * w) / 12.0, (h * h) / 12.0
    a = w2 * c * c + h2 * s * s
    b = w2 * s * s + h2 * c * c
    cc = (w2 - h2) * c * s
    return cx, cy, a, b, cc


def _body(feat_ref, scores_ref,
          ob_ref, os_ref, ol_ref,
          s4_ref, p4_ref, tab_ref, sel_ref, offs_ref, valid_ref, keep_ref):
    f32 = jnp.float32
    bf16 = jnp.bfloat16
    i32 = jnp.int32
    sc4 = scores_ref[...]
    key = jnp.where(sc4 > _SCORE_THRESH,
                    jax.lax.bitcast_convert_type(sc4, i32), i32(-1))
    gg = ((jax.lax.broadcasted_iota(i32, (_L, _L), 0) // 128)
          == (jax.lax.broadcasted_iota(i32, (_L, _L), 1) // 128)).astype(f32)
    dn_mv = (((1,), (0,)), ((), ()))

    def _gcount(mask):
        col = jnp.sum(mask.astype(f32), axis=0, keepdims=True)
        return jax.lax.dot_general(col, gg, dn_mv,
                                   preferred_element_type=f32)

    def _val_step(_, lohi):
        lo, hi = lohi
        d = jnp.maximum((hi - lo) // 4, 1)
        m1 = jnp.minimum(lo + d, hi)
        m2 = jnp.minimum(m1 + d, hi)
        m3 = jnp.minimum(m2 + d, hi)
        b1 = _gcount(key >= m1) >= _K
        b2 = _gcount(key >= m2) >= _K
        b3 = _gcount(key >= m3) >= _K
        lo = jnp.where(b3, m3, jnp.where(b2, m2, jnp.where(b1, m1, lo)))
        hi = jnp.where(b3, hi, jnp.where(b2, m3, jnp.where(b1, m2, m1)))
        return lo, hi

    tau, _ = jax.lax.fori_loop(
        0, 17, _val_step, (jnp.full((1, _L), -1, i32),
                           jnp.full((1, _L), _ONE_BITS, i32)))
    eq = key == tau
    m_need = f32(_K) - _gcount(key > tau)

    lstrict = (jax.lax.broadcasted_iota(i32, (_NB, _NB), 1)
               < jax.lax.broadcasted_iota(i32, (_NB, _NB), 0)).astype(f32)
    tri_s = ((jax.lax.broadcasted_iota(i32, (_L, _L), 0)
              < jax.lax.broadcasted_iota(i32, (_L, _L), 1))
             & ((jax.lax.broadcasted_iota(i32, (_L, _L), 0) // 128)
                == (jax.lax.broadcasted_iota(i32, (_L, _L), 1) // 128))
             ).astype(f32)
    eqf = eq.astype(f32)
    rowcnt = jax.lax.dot_general(eqf, gg, dn_mv,
                                 preferred_element_type=f32)
    rank2d = (jax.lax.dot_general(lstrict, rowcnt, dn_mv,
                                  preferred_element_type=f32)
              + jax.lax.dot_general(eqf, tri_s, dn_mv,
                                    preferred_element_type=f32))
    sel = ((key > tau) | (eq & (rank2d < m_need))).astype(f32)
    sel_ref[...] = sel

    cnts = jax.lax.dot_general(sel, gg, dn_mv,
                               preferred_element_type=f32)
    offs512 = jnp.minimum(jax.lax.dot_general(
        lstrict, cnts, dn_mv, preferred_element_type=f32), f32(_K))
    for i in range(_B):
        offs_ref[i] = offs512[:, 128 * i:128 * i + 1]
    tri = ((jax.lax.broadcasted_iota(i32, (_L, _L), 0)
            <= jax.lax.broadcasted_iota(i32, (_L, _L), 1))
           & ((jax.lax.broadcasted_iota(i32, (_L, _L), 0) // 128)
              == (jax.lax.broadcasted_iota(i32, (_L, _L), 1) // 128))
           ).astype(f32)
    riota = jax.lax.broadcasted_iota(i32, (128, _L), 0).astype(f32)
    dn = (((1,), (1,)), ((), ()))

    def _gather_step(t, carry):
        m = sel_ref[pl.ds(t, 1), :]
        csum = jax.lax.dot_general(m, tri, dn_mv,
                                   preferred_element_type=f32)
        g = jnp.where(m > 0.5, csum - 1.0, -1.0)
        ct = (riota == g).astype(f32)
        dt = feat_ref[:, pl.ds(t, 1), :].reshape(9, _L)
        for i in range(_B):
            compact = jax.lax.dot_general(
                ct[:, 128 * i:128 * (i + 1)], dt[:, 128 * i:128 * (i + 1)],
                dn, preferred_element_type=f32, precision=_HI)
            off = offs_ref[i, t, 0].astype(i32)
            tab_ref[i, pl.ds(off, 128), :] = compact
        return carry

    jax.lax.fori_loop(0, _NB, _gather_step, 0, unroll=16)

    def _build_step(img, carry):
        tab_ref[img, _K:, :] = jnp.zeros((_TPAD - _K, 9), f32)
        tab = tab_ref[img, 0:_KP, :]
        tabt = jnp.transpose(tab)
        cxc, cyc, wc, hc, ac = (tab[:, i:i + 1] for i in range(5))
        scc, labc, idxc = tab[:, 5:6], tab[:, 6:7], tab[:, 7:8]
        cxr, cyr, wr, hr, ar = (tabt[i:i + 1, :] for i in range(5))
        scr, labr, idxr = tabt[5:6, :], tabt[6:7, :], tabt[7:8, :]

        x1, y1, a1, b1, c1 = _gauss_params(cxc, cyc, wc, hc, ac)
        x2, y2, a2, b2, c2 = _gauss_params(cxr, cyr, wr, hr, ar)
        eps = 1e-7
        dx = x1 - x2
        dy = y1 - y2
        ab = (a1 + a2) * (b1 + b2) - (c1 + c2) ** 2
        t1 = 0.25 * ((a1 + a2) * dy * dy + (b1 + b2) * dx * dx) / (ab + eps)
        t2 = 0.5 * ((c1 + c2) * (-dx) * dy) / (ab + eps)
        d1 = jnp.clip(a1 * b1 - c1 * c1, eps, None)
        d2 = jnp.clip(a2 * b2 - c2 * c2, eps, None)
        t3 = 0.5 * jnp.log(ab / (4.0 * jnp.sqrt(d1 * d2) + eps) + eps)
        bd = jnp.clip(t1 + t2 + t3, eps, 100.0)
        hd = jnp.sqrt(1.0 - jnp.exp(-bd) + eps)
        iou = 1.0 - hd

        prio = (scc > scr) | ((scc == scr) & (idxc < idxr))
        p4_ref[img] = prio.astype(bf16)
        s4_ref[img] = ((iou > _NMS_THRESH) & (labc == labr)
                       & prio).astype(bf16)
        valid_ref[img] = (scr > _SCORE_THRESH).astype(bf16)
        return carry

    jax.lax.fori_loop(0, _B, _build_step, 0)

    valid4 = valid_ref[...]

    def _one_step(old):
        hits = [jax.lax.dot_general(old[i], s4_ref[i], dn_mv,
                                    preferred_element_type=f32
                                    ).reshape(1, 1, _KP)
                for i in range(_B)]
        hit4 = jnp.concatenate(hits, axis=0)
        return jnp.where(hit4 < 0.5, valid4, bf16(0))

    def _nms_cond(changed):
        return changed

    def _nms_step(_):
        old = keep_ref[...]
        new = _one_step(old)
        keep_ref[...] = new
        return jnp.sum(jnp.abs((new - old).astype(f32))) > 0.0

    keep_ref[...] = _one_step(_one_step(valid4))
    jax.lax.while_loop(_nms_cond, _nms_step, jnp.bool_(True))

    piota = jax.lax.broadcasted_iota(i32, (_DETP, _KP), 0).astype(f32)

    def _out_step(img, carry):
        kept = keep_ref[img]
        rank = jax.lax.dot_general(kept, p4_ref[img], dn_mv,
                                   preferred_element_type=f32)
        oh = ((piota == rank) & (kept > bf16(0.5))).astype(f32)
        tab = tab_ref[img, 0:_KP, :]
        out = jax.lax.dot_general(oh, tab, dn_mv,
                                  preferred_element_type=f32,
                                  precision=_HI)
        outt = jax.lax.dot_general(jnp.transpose(tab), oh, dn,
                                   preferred_element_type=f32,
                                   precision=_HI)
        ob_ref[pl.ds(img, 1), :, :] = out[:_DET, 0:5].reshape(1, _DET, 5)
        os_ref[pl.ds(img, 1), :] = outt[5:6, :_DET]
        ind = outt[8:9, :_DET]
        ol_ref[pl.ds(img, 1), :] = jnp.where(
            ind > 0.5, outt[6:7, :_DET], -1.0).astype(jnp.int32)
        return carry

    jax.lax.fori_loop(0, _B, _out_step, 0)


@jax.jit
def kernel(boxes, scores, labels):
    b = boxes.shape[0]
    f32 = jnp.float32
    pad = _NP - _N
    sc = jnp.pad(scores, ((0, 0), (0, pad)),
                 constant_values=-1.0).reshape(b, _NB, 128)
    sc = jnp.transpose(sc, (1, 0, 2)).reshape(_NB, b * 128)
    bx = jnp.pad(boxes, ((0, 0), (0, pad), (0, 0)))
    bx = jnp.transpose(bx, (0, 2, 1))
    scp = jnp.pad(scores, ((0, 0), (0, pad)))[:, None, :]
    lbp = jnp.pad(labels, ((0, 0), (0, pad))).astype(f32)[:, None, :]
    idx = jnp.broadcast_to(jnp.arange(_NP, dtype=f32), (b, 1, _NP))
    one = jnp.ones((b, 1, _NP), f32)
    feat = jnp.concatenate([bx, scp, lbp, idx, one],
                           axis=1).reshape(b, 9, _NB, 128)
    feat = jnp.transpose(feat, (1, 2, 0, 3)).reshape(9, _NB, b * 128)

    out = pl.pallas_call(
        _body,
        out_shape=[
            jax.ShapeDtypeStruct((b, _DET, 5), f32),
            jax.ShapeDtypeStruct((b, _DET), f32),
            jax.ShapeDtypeStruct((b, _DET), jnp.int32),
        ],
        scratch_shapes=[
            pltpu.VMEM((_B, _KP, _KP), jnp.bfloat16),
            pltpu.VMEM((_B, _KP, _KP), jnp.bfloat16),
            pltpu.VMEM((_B, _TPAD, 9), f32),
            pltpu.VMEM((_NB, _L), f32),
            pltpu.VMEM((_B, _NB, 1), f32),
            pltpu.VMEM((_B, 1, _KP), jnp.bfloat16),
            pltpu.VMEM((_B, 1, _KP), jnp.bfloat16),
        ],
    )(feat, sc)
    return tuple(out)
